# baseline (speedup 1.0000x reference)
import jax
import jax.numpy as jnp
from jax.experimental import pallas as pl
from jax.experimental.pallas import tpu as pltpu

_S = 2048
_NIN = 4
_NBUF = 3
_f32 = jnp.float32


def _body(x_hbm, gt_ref, w_ref, o_hbm, xbuf, in_sems, out_sems):
    b = pl.program_id(0)
    B = pl.num_programs(0)
    C = x_hbm.shape[1]
    HW = x_hbm.shape[2]
    K = w_ref.shape[1]
    nout = HW // _S
    cin = C // _NIN
    p = jax.lax.rem(b, _NBUF)
    pn = jax.lax.rem(b + 1, _NBUF)

    def in_copy(batch, parity, j):
        return pltpu.make_async_copy(
            x_hbm.at[batch, pl.ds(j * cin, cin), :],
            xbuf.at[parity, pl.ds(j * cin, cin), :],
            in_sems.at[parity, j])

    def out_copy(batch, parity, i):
        return pltpu.make_async_copy(
            xbuf.at[parity, :, pl.ds(i * _S, _S)],
            o_hbm.at[batch, :, pl.ds(i * _S, _S)],
            out_sems.at[parity, i])

    @pl.when(b == 0)
    def _():
        for j in range(_NIN):
            in_copy(0, 0, j).start()

    for j in range(_NIN):
        in_copy(b, p, j).wait()

    @pl.when(b >= 2)
    def _():
        for i in range(nout):
            out_copy(b - 2, pn, i).wait()

    @pl.when(b + 1 < B)
    def _():
        for j in range(_NIN):
            in_copy(b + 1, pn, j).start()

    cls = jax.lax.broadcasted_iota(jnp.int32, (K, HW), 0)
    oh_full = (cls == gt_ref[b]).astype(_f32)

    fsum = jnp.zeros((K, C), _f32)
    fsq = jnp.zeros((K, C), _f32)
    for i in range(nout):
        oh = oh_full[:, i * _S:(i + 1) * _S]
        xc = xbuf[p, :, pl.ds(i * _S, _S)]
        xcb = xc.astype(jnp.bfloat16)
        fsum = fsum + jax.lax.dot_general(
            oh, xc, (((1,), (1,)), ((), ())), preferred_element_type=_f32)
        fsq = fsq + jax.lax.dot_general(
            oh.astype(jnp.bfloat16), xcb * xcb, (((1,), (1,)), ((), ())),
            preferred_element_type=_f32)
    cnt = jnp.sum(oh_full, axis=1, keepdims=True)

    rc = 1.0 / jnp.where(cnt > 0, cnt, 1.0)
    mean = fsum * rc
    var = jnp.maximum(fsq * rc - mean * mean, 0.0)
    std = jnp.sqrt(var) + 1e-7
    wm = w_ref[b]
    hp = jax.lax.Precision.HIGHEST
    sm = jax.lax.dot_general(
        wm, mean, (((1,), (0,)), ((), ())), precision=hp,
        preferred_element_type=_f32)
    ss = jax.lax.dot_general(
        wm, std, (((1,), (0,)), ((), ())), precision=hp,
        preferred_element_type=_f32)
    rss = ss / std
    off = sm - mean * rss

    def gather(tbl, oh):
        return jax.lax.dot_general(
            tbl, oh, (((0,), (0,)), ((), ())),
            preferred_element_type=_f32)

    for i in range(nout):
        oh = oh_full[:, i * _S:(i + 1) * _S]
        xc = xbuf[p, :, pl.ds(i * _S, _S)]
        rg = gather(rss, oh)
        og = gather(off, oh)
        xbuf[p, :, pl.ds(i * _S, _S)] = xc * rg + og
        out_copy(b, p, i).start()

    @pl.when(b == B - 1)
    def _():
        for i in range(nout):
            out_copy(b - 1, jax.lax.rem(b - 1, _NBUF), i).wait()
        for i in range(nout):
            out_copy(b, p, i).wait()


def kernel(x, gt, aug_rand_info):
    B, C, H, W = x.shape
    K = aug_rand_info.shape[1]
    HW = H * W
    xf = x.reshape(B, C, HW)
    gtf = gt.reshape(B, 1, HW).astype(jnp.int32)
    w = aug_rand_info.reshape(B, K, K)
    xs = pl.pallas_call(
        _body,
        grid=(B,),
        in_specs=[
            pl.BlockSpec(memory_space=pl.ANY),
            pl.BlockSpec(memory_space=pltpu.MemorySpace.VMEM),
            pl.BlockSpec(memory_space=pltpu.MemorySpace.VMEM),
        ],
        out_specs=pl.BlockSpec(memory_space=pl.ANY),
        out_shape=jax.ShapeDtypeStruct((B, C, HW), x.dtype),
        scratch_shapes=[
            pltpu.VMEM((_NBUF, C, HW), _f32),
            pltpu.SemaphoreType.DMA((_NBUF, _NIN)),
            pltpu.SemaphoreType.DMA((_NBUF, HW // _S)),
        ],
    )(xf, gtf, w)
    return (x, xs.reshape(B, C, H, W))
